# baseline (speedup 1.0000x reference)
import jax
import jax.numpy as jnp
from jax.experimental import pallas as pl

TOKEN_BLOCK = 1024


def _router_kernel(x_ref, wt_ref, b_ref, w_out_ref, l_out_ref):
    logits = jax.lax.dot_general(
        x_ref[...], wt_ref[...],
        dimension_numbers=(((1,), (0,)), ((), ())),
        preferred_element_type=jnp.float32,
    ) + b_ref[...]
    l_out_ref[...] = logits
    m = jnp.max(logits, axis=1, keepdims=True)
    e = jnp.exp(logits - m)
    s = jnp.sum(e, axis=1, keepdims=True)
    w_out_ref[...] = e / s


def kernel(x, W, b):
    tokens, feat = x.shape
    n_exp = W.shape[0]
    wt = W.T
    b2 = b.reshape(1, n_exp)
    grid = (tokens // TOKEN_BLOCK,)
    weights, logits = pl.pallas_call(
        _router_kernel,
        grid=grid,
        in_specs=[
            pl.BlockSpec((TOKEN_BLOCK, feat), lambda i: (i, 0)),
            pl.BlockSpec((feat, n_exp), lambda i: (0, 0)),
            pl.BlockSpec((1, n_exp), lambda i: (0, 0)),
        ],
        out_specs=[
            pl.BlockSpec((TOKEN_BLOCK, n_exp), lambda i: (i, 0)),
            pl.BlockSpec((TOKEN_BLOCK, n_exp), lambda i: (i, 0)),
        ],
        out_shape=[
            jax.ShapeDtypeStruct((tokens, n_exp), jnp.float32),
            jax.ShapeDtypeStruct((tokens, n_exp), jnp.float32),
        ],
    )(x, wt, b2)
    return (weights, logits)
